# speedup vs baseline: 4.5104x; 4.5104x over previous
import functools

import jax
import jax.numpy as jnp
from jax import lax
from jax.experimental import pallas as pl
from jax.experimental.pallas import tpu as pltpu
from jax.experimental.pallas import tpu_sc as plsc

N = 10000
E = 320000
D = 128
NUM_LAYERS = 5

NC = 2
NS = 16
NW = NC * NS
CH = 128
CPT = 79
EPT = CPT * CH
E_PAD = NW * EPT
ROWS_PER_TILE = 640
AGG_ROWS = NS * ROWS_PER_TILE


def _sc_agg_body(h_hbm, src_hbm, dst_hbm, zeros_hbm, out_hbm,
                 agg_sh, src_v, dst_v, rows_v, sem):
    c = lax.axis_index("c")
    s = lax.axis_index("s")
    wid = c * NS + s

    pltpu.sync_copy(zeros_hbm, agg_sh.at[pl.ds(s * ROWS_PER_TILE, ROWS_PER_TILE)])

    pltpu.sync_copy(src_hbm.at[wid], src_v)
    pltpu.sync_copy(dst_hbm.at[wid], dst_v)
    plsc.subcore_barrier()

    @pl.loop(0, CPT)
    def _(ci):
        pltpu.async_copy(h_hbm.at[src_v.at[ci]], rows_v, sem).wait()
        pltpu.sync_copy(rows_v, agg_sh.at[dst_v.at[ci]], add=True)

    plsc.subcore_barrier()
    pltpu.sync_copy(agg_sh.at[pl.ds(s * ROWS_PER_TILE, ROWS_PER_TILE)],
                    out_hbm.at[c, pl.ds(s * ROWS_PER_TILE, ROWS_PER_TILE)])


_sc_agg = pl.kernel(
    _sc_agg_body,
    out_type=jax.ShapeDtypeStruct((NC, AGG_ROWS, D), jnp.float32),
    mesh=plsc.VectorSubcoreMesh(core_axis_name="c", subcore_axis_name="s",
                                num_cores=NC, num_subcores=NS),
    scratch_types=[
        pltpu.VMEM_SHARED((AGG_ROWS, D), jnp.float32),
        pltpu.VMEM((CPT, CH), jnp.int32),
        pltpu.VMEM((CPT, CH), jnp.int32),
        pltpu.VMEM((CH, D), jnp.float32),
        pltpu.SemaphoreType.DMA,
    ],
)


def _encoder_body(x_ref, w_ref, b_ref, o_ref):
    o_ref[...] = (
        jnp.dot(x_ref[...], w_ref[...], preferred_element_type=jnp.float32)
        + b_ref[...]
    )


_encoder = pl.pallas_call(
    _encoder_body,
    out_shape=jax.ShapeDtypeStruct((N, D), jnp.float32),
)


def _layer_body(h_ref, a_ref, w1_ref, b1_ref, w2_ref, b2_ref, o_ref, *, last):
    z = h_ref[...] + a_ref[0, :N, :] + a_ref[1, :N, :]
    z = jnp.dot(z, w1_ref[...], preferred_element_type=jnp.float32) + b1_ref[...]
    z = jnp.maximum(z, 0.0)
    z = jnp.dot(z, w2_ref[...], preferred_element_type=jnp.float32) + b2_ref[...]
    if not last:
        z = jnp.maximum(z, 0.0)
    o_ref[...] = z


def _layer_call(last):
    return pl.pallas_call(
        functools.partial(_layer_body, last=last),
        out_shape=jax.ShapeDtypeStruct((N, D), jnp.float32),
    )


_layer_mid = _layer_call(False)
_layer_last = _layer_call(True)


@jax.jit
def _run(x, src_p, dst_p, W_in, b_in, W1, b1, W2, b2):
    zeros = jnp.zeros((ROWS_PER_TILE, D), jnp.float32)
    h = _encoder(x, W_in, b_in.reshape(1, D))
    for l in range(NUM_LAYERS):
        agg = _sc_agg(h, src_p, dst_p, zeros)
        layer = _layer_last if l == NUM_LAYERS - 1 else _layer_mid
        h = layer(h, agg, W1[l], b1[l].reshape(1, D), W2[l], b2[l].reshape(1, D))
    return h


def kernel(x, edge_index, W_in, b_in, W1, b1, W2, b2):
    src = edge_index[0].astype(jnp.int32)
    dst = edge_index[1].astype(jnp.int32)
    pad = E_PAD - E
    src_p = jnp.concatenate([src, jnp.zeros((pad,), jnp.int32)]).reshape(NW, CPT, CH)
    dst_p = jnp.concatenate([dst, jnp.full((pad,), N, jnp.int32)]).reshape(NW, CPT, CH)
    return _run(x, src_p, dst_p, W_in, b_in, W1, b1, W2, b2)
